# Initial kernel scaffold; baseline (speedup 1.0000x reference)
import jax
import jax.numpy as jnp
from jax.experimental import pallas as pl


def kernel(x, edge_index, weights, W_gcn, W_sage_self, W_sage_neigh, W_smax_self, W_smax_neigh, W_gin):
    raise NotImplementedError("write your pallas kernel here")


# speedup vs baseline: 1.7453x; 1.7453x over previous
import jax
import jax.numpy as jnp
from jax.experimental import pallas as pl
from jax.experimental.pallas import tpu as pltpu

_BLK = 1000


def _mix_body(w_ref, x_ref, ssum_ref, wsum_ref, smax_ref, degc_ref,
              wg_ref, wss_ref, wsn_ref, wxs_ref, wxn_ref, wgin_ref,
              out_ref):
    x = x_ref[...]
    ssum = ssum_ref[...]
    wsum = wsum_ref[...]
    smax = smax_ref[...]
    degc = degc_ref[...]
    dis = jax.lax.rsqrt(degc)
    smax = jnp.where(smax > -3e38, smax, 0.0)

    def dot(a, w):
        return jax.lax.dot_general(a, w, (((1,), (0,)), ((), ())),
                                   preferred_element_type=jnp.float32)

    gcn = dot(wsum, wg_ref[...]) * dis
    sage = dot(x, wss_ref[...]) + dot(ssum, wsn_ref[...]) / degc
    smax_o = dot(x, wxs_ref[...]) + dot(smax, wxn_ref[...])
    gin = dot(x + ssum, wgin_ref[...])

    def elu(v):
        return jnp.where(v > 0, v, jnp.exp(jnp.minimum(v, 0.0)) - 1.0)

    out_ref[...] = (w_ref[0] * elu(gcn) + w_ref[1] * elu(sage)
                    + w_ref[2] * elu(smax_o) + w_ref[3] * elu(gin))


def kernel(x, edge_index, weights, W_gcn, W_sage_self, W_sage_neigh,
           W_smax_self, W_smax_neigh, W_gin):
    n, d = x.shape
    src = edge_index[0]
    dst = edge_index[1]

    ones = jnp.ones(src.shape, dtype=x.dtype)
    deg = jax.ops.segment_sum(ones, dst, num_segments=n)
    deg_c = jnp.maximum(deg, 1.0)
    dis = jax.lax.rsqrt(deg_c)

    xs = x[src]
    ssum = jax.ops.segment_sum(xs, dst, num_segments=n)
    wsum = jax.ops.segment_sum(xs * dis[src][:, None], dst, num_segments=n)
    smax = jax.ops.segment_max(xs, dst, num_segments=n)

    degc2 = deg_c[:, None]

    row = lambda i: (i, 0)
    full = lambda i: (0, 0)
    grid = (n // _BLK,)
    rowspec = pl.BlockSpec((_BLK, d), row)
    wspec = pl.BlockSpec((d, d), full)

    return pl.pallas_call(
        _mix_body,
        grid=grid,
        in_specs=[
            pl.BlockSpec(memory_space=pltpu.SMEM),
            rowspec, rowspec, rowspec, rowspec,
            pl.BlockSpec((_BLK, 1), row),
            wspec, wspec, wspec, wspec, wspec, wspec,
        ],
        out_specs=rowspec,
        out_shape=jax.ShapeDtypeStruct((n, d), x.dtype),
    )(weights, x, ssum, wsum, smax, degc2,
      W_gcn, W_sage_self, W_sage_neigh, W_smax_self, W_smax_neigh, W_gin)
